# Initial kernel scaffold; baseline (speedup 1.0000x reference)
import jax
import jax.numpy as jnp
from jax.experimental import pallas as pl


def kernel(x, edge_index, edge_type, att0, basis0, q0, k0, b0, att1, basis1, q1, k1, b1, att2, basis2, q2, k2, b2):
    raise NotImplementedError("write your pallas kernel here")


# speedup vs baseline: 19.7590x; 19.7590x over previous
import jax
import jax.numpy as jnp
from jax import lax
from jax.experimental import pallas as pl
from jax.experimental.pallas import tpu as pltpu
from jax.experimental.pallas import tpu_sc as plsc

_N = 10000
_E = 320000
_R = 8
_NBASES = 4
_D = 128
_NEG = 0.2

_NC = 2
_NS = 16
_NW = _NC * _NS
_EPW = _E // _NW
_CH = 128
_NCH = (_EPW + _CH - 1) // _CH
_EPAD = _NCH * _CH
_TAIL = _EPW - (_NCH - 1) * _CH
_LANE = 16

_NBLK = 512
_GRID = (_N + _NBLK - 1) // _NBLK

_SW = 632
_SWL = _N - _SW * (_NS - 1)
_DNP = 10112


def _dense_core(h, att_ref, basis_ref, q_ref, k_ref, xw_ref, sq_ref, sk_ref):
    q1 = q_ref[...]
    k1 = k_ref[...]
    dn = (((1,), (1,)), ((), ()))
    for r in range(_R):
        w_r = att_ref[r, 0] * basis_ref[0]
        for b in range(1, _NBASES):
            w_r = w_r + att_ref[r, b] * basis_ref[b]
        xw_r = jnp.dot(h, w_r, preferred_element_type=jnp.float32,
                       precision=lax.Precision.HIGHEST)
        xw_ref[r] = xw_r
        sq_ref[r:r + 1, :] = lax.dot_general(
            q1, xw_r, dn, preferred_element_type=jnp.float32,
            precision=lax.Precision.HIGHEST)
        sk_ref[r:r + 1, :] = lax.dot_general(
            k1, xw_r, dn, preferred_element_type=jnp.float32,
            precision=lax.Precision.HIGHEST)


def _dense0_body(x_ref, att_ref, basis_ref, q_ref, k_ref,
                 xw_ref, sq_ref, sk_ref):
    _dense_core(x_ref[...], att_ref, basis_ref, q_ref, k_ref,
                xw_ref, sq_ref, sk_ref)


def _dense1_body(p_ref, b_ref, att_ref, basis_ref, q_ref, k_ref,
                 xw_ref, sq_ref, sk_ref):
    h = jnp.maximum(p_ref[0] + p_ref[1] + b_ref[...], 0.0)
    _dense_core(h, att_ref, basis_ref, q_ref, k_ref, xw_ref, sq_ref, sk_ref)


_DENSE_OUT = [
    jax.ShapeDtypeStruct((_R, _N, _D), jnp.float32),
    jax.ShapeDtypeStruct((_R, _N), jnp.float32),
    jax.ShapeDtypeStruct((_R, _N), jnp.float32),
]
_DENSE_OUT_SPECS = [
    pl.BlockSpec((_R, _NBLK, _D), lambda i: (0, i, 0)),
    pl.BlockSpec((_R, _NBLK), lambda i: (0, i)),
    pl.BlockSpec((_R, _NBLK), lambda i: (0, i)),
]
_W_SPECS = [
    pl.BlockSpec((_R, _NBASES), lambda i: (0, 0), memory_space=pltpu.SMEM),
    pl.BlockSpec((_NBASES, _D, _D), lambda i: (0, 0, 0)),
    pl.BlockSpec((1, _D), lambda i: (0, 0)),
    pl.BlockSpec((1, _D), lambda i: (0, 0)),
]


def _dense0(x, att, basis, q1, k1):
    return pl.pallas_call(
        _dense0_body,
        grid=(_GRID,),
        in_specs=[pl.BlockSpec((_NBLK, _D), lambda i: (i, 0))] + _W_SPECS,
        out_specs=_DENSE_OUT_SPECS,
        out_shape=_DENSE_OUT,
    )(x, att, basis, q1, k1)


def _dense1(p, b1, att, basis, q1, k1):
    return pl.pallas_call(
        _dense1_body,
        grid=(_GRID,),
        in_specs=[pl.BlockSpec((_NC, _NBLK, _D), lambda i: (0, i, 0)),
                  pl.BlockSpec((1, _D), lambda i: (0, 0))] + _W_SPECS,
        out_specs=_DENSE_OUT_SPECS,
        out_shape=_DENSE_OUT,
    )(p, b1, att, basis, q1, k1)


def _combine_body(p_ref, b_ref, o_ref):
    o_ref[...] = p_ref[0] + p_ref[1] + b_ref[...]


def _combine(p, b1):
    return pl.pallas_call(
        _combine_body,
        grid=(_GRID,),
        in_specs=[pl.BlockSpec((_NC, _NBLK, _D), lambda i: (0, i, 0)),
                  pl.BlockSpec((1, _D), lambda i: (0, 0))],
        out_specs=pl.BlockSpec((_NBLK, _D), lambda i: (i, 0)),
        out_shape=jax.ShapeDtypeStruct((_N, _D), jnp.float32),
    )(p, b1)


_MESH = plsc.VectorSubcoreMesh(
    core_axis_name="c", subcore_axis_name="s",
    num_cores=_NC, num_subcores=_NS)


def _worker_base():
    cid = lax.axis_index("c")
    sid = lax.axis_index("s")
    wid = sid * _NC + cid
    base = pl.multiple_of(wid * _EPW, 8)
    return cid, sid, base


def _s1_body(sqf, skf, srcg, dstg, et, ex_out, den_out,
             table, eta, nbuf, acc, dst2d, den_s):
    cid, sid, base = _worker_base()
    zi = jnp.zeros((_LANE,), jnp.int32)
    zf = jnp.zeros((_LANE,), jnp.float32)

    pltpu.sync_copy(et.at[pl.ds(base, _EPW)], eta.at[pl.ds(0, _EPW)])
    pltpu.sync_copy(dstg.at[pl.ds(base, _EPW)], nbuf.at[pl.ds(0, _EPW)])
    for t in range(_EPW // _LANE, _EPAD // _LANE):
        eta[pl.ds(t * _LANE, _LANE)] = zi
        nbuf[pl.ds(t * _LANE, _LANE)] = zi

    pltpu.sync_copy(sqf, table)

    @pl.loop(0, _EPAD // _LANE)
    def _p1(i):
        o = pl.multiple_of(i * _LANE, _LANE)
        idx = eta[pl.ds(o, _LANE)] * _N + nbuf[pl.ds(o, _LANE)]
        acc[pl.ds(o, _LANE)] = plsc.load_gather(table, [idx])

    pltpu.sync_copy(srcg.at[pl.ds(base, _EPW)], nbuf.at[pl.ds(0, _EPW)])
    pltpu.sync_copy(skf, table)

    @pl.loop(0, _EPAD // _LANE)
    def _p2(i):
        o = pl.multiple_of(i * _LANE, _LANE)
        idx = eta[pl.ds(o, _LANE)] * _N + nbuf[pl.ds(o, _LANE)]
        kj = plsc.load_gather(table, [idx])
        z = acc[pl.ds(o, _LANE)] + kj
        al = jnp.maximum(z, 0.0) + _NEG * jnp.minimum(z, 0.0)
        acc[pl.ds(o, _LANE)] = jnp.exp(al)

    pltpu.sync_copy(acc.at[pl.ds(0, _EPW)], ex_out.at[pl.ds(base, _EPW)])
    for t in range(_EPW // _LANE, _EPAD // _LANE):
        acc[pl.ds(t * _LANE, _LANE)] = zf

    @pl.loop(0, _NCH - 1)
    def _ld(j):
        off = pl.multiple_of(base + j * _CH, 8)
        pltpu.sync_copy(dstg.at[pl.ds(off, _CH)], dst2d.at[j])
    lastoff = pl.multiple_of(base + (_NCH - 1) * _CH, 8)
    pltpu.sync_copy(dstg.at[pl.ds(lastoff, _TAIL)],
                    dst2d.at[_NCH - 1, pl.ds(0, _TAIL)])
    for t in range(_TAIL // _LANE, _CH // _LANE):
        dst2d[_NCH - 1, pl.ds(t * _LANE, _LANE)] = zi

    @pl.when(sid == 0)
    def _z():
        @pl.loop(0, _DNP // _LANE)
        def _zz(i):
            table[pl.ds(i * _LANE, _LANE)] = zf
        pltpu.sync_copy(table.at[pl.ds(0, _DNP)], den_s)

    plsc.subcore_barrier()

    @pl.loop(0, _NCH)
    def _sc(j):
        pltpu.sync_copy(acc.at[pl.ds(j * _CH, _CH)],
                        den_s.at[dst2d.at[j]], add=True)

    plsc.subcore_barrier()

    @pl.when(sid == 0)
    def _exp():
        pltpu.sync_copy(den_s, den_out.at[cid])


def _s1(sqf, skf, srcg, dstg, et):
    f = pl.kernel(
        _s1_body,
        out_type=[
            jax.ShapeDtypeStruct((_E,), jnp.float32),
            jax.ShapeDtypeStruct((_NC, _DNP), jnp.float32),
        ],
        mesh=_MESH,
        compiler_params=pltpu.CompilerParams(needs_layout_passes=False),
        scratch_types=[
            pltpu.VMEM((_R * _N,), jnp.float32),
            pltpu.VMEM((_EPAD,), jnp.int32),
            pltpu.VMEM((_EPAD,), jnp.int32),
            pltpu.VMEM((_EPAD,), jnp.float32),
            pltpu.VMEM((_NCH, _CH), jnp.int32),
            pltpu.VMEM_SHARED((_DNP,), jnp.float32),
        ],
    )
    return f(sqf, skf, srcg, dstg, et)


def _s2_body(xwf, ex, denp, srcg, dstg, et, out_p,
             den, dst2d, av, rb, tbuf, sbuf, xbuf, gidxr, outacc):
    cid, sid, base = _worker_base()
    zi = jnp.zeros((_LANE,), jnp.int32)
    zf = jnp.zeros((_LANE,), jnp.float32)

    pltpu.sync_copy(denp.at[0], den)
    pltpu.sync_copy(denp.at[1], av.at[pl.ds(0, _DNP)])

    @pl.loop(0, _DNP // _LANE)
    def _dsum(i):
        o = pl.multiple_of(i * _LANE, _LANE)
        den[pl.ds(o, _LANE)] = den[pl.ds(o, _LANE)] + av[pl.ds(o, _LANE)]

    @pl.loop(0, _NCH - 1)
    def _ld(j):
        off = pl.multiple_of(base + j * _CH, 8)
        pltpu.sync_copy(dstg.at[pl.ds(off, _CH)], dst2d.at[j])
    lastoff = pl.multiple_of(base + (_NCH - 1) * _CH, 8)
    pltpu.sync_copy(dstg.at[pl.ds(lastoff, _TAIL)],
                    dst2d.at[_NCH - 1, pl.ds(0, _TAIL)])
    for t in range(_TAIL // _LANE, _CH // _LANE):
        dst2d[_NCH - 1, pl.ds(t * _LANE, _LANE)] = zi

    @pl.loop(0, _NCH - 1)
    def _aw(j):
        off = pl.multiple_of(base + j * _CH, 8)
        pltpu.sync_copy(ex.at[pl.ds(off, _CH)], xbuf)
        for s in range(_CH // _LANE):
            d = dst2d[j, pl.ds(s * _LANE, _LANE)]
            dn = plsc.load_gather(den, [d])
            av[pl.ds(j * _CH + s * _LANE, _LANE)] = (
                xbuf[pl.ds(s * _LANE, _LANE)] / (dn + 1e-16))
    pltpu.sync_copy(ex.at[pl.ds(lastoff, _TAIL)], xbuf.at[pl.ds(0, _TAIL)])
    for s in range(_TAIL // _LANE):
        d = dst2d[_NCH - 1, pl.ds(s * _LANE, _LANE)]
        dn = plsc.load_gather(den, [d])
        av[pl.ds((_NCH - 1) * _CH + s * _LANE, _LANE)] = (
            xbuf[pl.ds(s * _LANE, _LANE)] / (dn + 1e-16))
    for t in range(_EPW // _LANE, _EPAD // _LANE):
        av[pl.ds(t * _LANE, _LANE)] = zf

    @pl.loop(0, _CH)
    def _zr(r):
        for s in range(_D // _LANE):
            rb[r, pl.ds(s * _LANE, _LANE)] = zf

    stripe = pl.multiple_of(sid * _SW, 8)
    for jj in range(4):
        pltpu.sync_copy(rb, outacc.at[pl.ds(stripe + jj * _CH, _CH)])

    @pl.when(sid < _NS - 1)
    def _zt0():
        pltpu.sync_copy(rb.at[pl.ds(0, _SW - 4 * _CH)],
                        outacc.at[pl.ds(stripe + 4 * _CH, _SW - 4 * _CH)])

    @pl.when(sid == _NS - 1)
    def _zt1():
        pltpu.sync_copy(rb.at[pl.ds(0, _SWL - 4 * _CH)],
                        outacc.at[pl.ds(stripe + 4 * _CH, _SWL - 4 * _CH)])

    plsc.subcore_barrier()

    def _do_chunk(j):
        pltpu.sync_copy(xwf.at[gidxr], rb)

        @pl.loop(0, _CH)
        def _row(r):
            abc = plsc.load_gather(av, [jnp.full((_LANE,), j * _CH + r,
                                                 jnp.int32)])
            for s in range(_D // _LANE):
                rb[r, pl.ds(s * _LANE, _LANE)] = (
                    rb[r, pl.ds(s * _LANE, _LANE)] * abc)

        pltpu.sync_copy(rb, outacc.at[dst2d.at[j]], add=True)

    @pl.loop(0, _NCH - 1)
    def _main(j):
        off = pl.multiple_of(base + j * _CH, 8)
        pltpu.sync_copy(et.at[pl.ds(off, _CH)], tbuf)
        pltpu.sync_copy(srcg.at[pl.ds(off, _CH)], sbuf)
        for s in range(_CH // _LANE):
            o = s * _LANE
            gidxr[pl.ds(o, _LANE)] = (
                tbuf[pl.ds(o, _LANE)] * _N + sbuf[pl.ds(o, _LANE)])
        _do_chunk(j)

    pltpu.sync_copy(et.at[pl.ds(lastoff, _TAIL)], tbuf.at[pl.ds(0, _TAIL)])
    pltpu.sync_copy(srcg.at[pl.ds(lastoff, _TAIL)], sbuf.at[pl.ds(0, _TAIL)])
    for s in range(_TAIL // _LANE):
        o = s * _LANE
        gidxr[pl.ds(o, _LANE)] = (
            tbuf[pl.ds(o, _LANE)] * _N + sbuf[pl.ds(o, _LANE)])
    for t in range(_TAIL // _LANE, _CH // _LANE):
        gidxr[pl.ds(t * _LANE, _LANE)] = zi
    _do_chunk(_NCH - 1)

    plsc.subcore_barrier()

    for jj in range(4):
        off = pl.multiple_of(stripe + jj * _CH, 8)
        pltpu.sync_copy(outacc.at[pl.ds(off, _CH)],
                        out_p.at[cid, pl.ds(off, _CH)])
    toff = pl.multiple_of(stripe + 4 * _CH, 8)

    @pl.when(sid < _NS - 1)
    def _ex0():
        pltpu.sync_copy(outacc.at[pl.ds(toff, _SW - 4 * _CH)],
                        out_p.at[cid, pl.ds(toff, _SW - 4 * _CH)])

    @pl.when(sid == _NS - 1)
    def _ex1():
        pltpu.sync_copy(outacc.at[pl.ds(toff, _SWL - 4 * _CH)],
                        out_p.at[cid, pl.ds(toff, _SWL - 4 * _CH)])


def _s2(xwf, ex, denp, srcg, dstg, et):
    f = pl.kernel(
        _s2_body,
        out_type=jax.ShapeDtypeStruct((_NC, _N, _D), jnp.float32),
        mesh=_MESH,
        compiler_params=pltpu.CompilerParams(needs_layout_passes=False),
        scratch_types=[
            pltpu.VMEM((_DNP,), jnp.float32),
            pltpu.VMEM((_NCH, _CH), jnp.int32),
            pltpu.VMEM((_EPAD,), jnp.float32),
            pltpu.VMEM((_CH, _D), jnp.float32),
            pltpu.VMEM((_CH,), jnp.int32),
            pltpu.VMEM((_CH,), jnp.int32),
            pltpu.VMEM((_CH,), jnp.float32),
            pltpu.VMEM((_CH,), jnp.int32),
            pltpu.VMEM_SHARED((_N, _D), jnp.float32),
        ],
    )
    return f(xwf, ex, denp, srcg, dstg, et)


def _layer0(x, srcg, dstg, et, att, basis, q, k):
    xw, sq, sk = _dense0(x, att, basis, q.reshape(1, _D), k.reshape(1, _D))
    ex, denp = _s1(sq.reshape(-1), sk.reshape(-1), srcg, dstg, et)
    return _s2(xw.reshape(_R * _N, _D), ex, denp, srcg, dstg, et)


def _layer(p, bprev, srcg, dstg, et, att, basis, q, k):
    xw, sq, sk = _dense1(p, bprev.reshape(1, _D), att, basis,
                         q.reshape(1, _D), k.reshape(1, _D))
    ex, denp = _s1(sq.reshape(-1), sk.reshape(-1), srcg, dstg, et)
    return _s2(xw.reshape(_R * _N, _D), ex, denp, srcg, dstg, et)


def kernel(x, edge_index, edge_type, att0, basis0, q0, k0, b0,
           att1, basis1, q1, k1, b1, att2, basis2, q2, k2, b2):
    srcg = edge_index[0]
    dstg = edge_index[1]
    et = edge_type
    p = _layer0(x, srcg, dstg, et, att0, basis0, q0, k0)
    p = _layer(p, b0, srcg, dstg, et, att1, basis1, q1, k1)
    p = _layer(p, b1, srcg, dstg, et, att2, basis2, q2, k2)
    return _combine(p, b2.reshape(1, _D))


# speedup vs baseline: 20.1433x; 1.0194x over previous
import jax
import jax.numpy as jnp
from jax import lax
from jax.experimental import pallas as pl
from jax.experimental.pallas import tpu as pltpu
from jax.experimental.pallas import tpu_sc as plsc

_N = 10000
_E = 320000
_R = 8
_NBASES = 4
_D = 128
_NEG = 0.2

_NC = 2
_NS = 16
_NW = _NC * _NS
_EPW = _E // _NW
_CH = 128
_NCH = (_EPW + _CH - 1) // _CH
_EPAD = _NCH * _CH
_TAIL = _EPW - (_NCH - 1) * _CH
_LANE = 16

_NBLK = 512
_GRID = (_N + _NBLK - 1) // _NBLK

_SW = 632
_SWL = _N - _SW * (_NS - 1)
_DNP = 10112

_CH2 = 64
_NCH2 = (_EPW + _CH2 - 1) // _CH2
_TAIL2 = _EPW - (_NCH2 - 1) * _CH2


def _dense_core(h, att_ref, basis_ref, q_ref, k_ref, xw_ref, sq_ref, sk_ref):
    q1 = q_ref[...]
    k1 = k_ref[...]
    dn = (((1,), (1,)), ((), ()))
    for r in range(_R):
        w_r = att_ref[r, 0] * basis_ref[0]
        for b in range(1, _NBASES):
            w_r = w_r + att_ref[r, b] * basis_ref[b]
        xw_r = jnp.dot(h, w_r, preferred_element_type=jnp.float32,
                       precision=lax.Precision.HIGHEST)
        xw_ref[r] = xw_r
        sq_ref[r:r + 1, :] = lax.dot_general(
            q1, xw_r, dn, preferred_element_type=jnp.float32,
            precision=lax.Precision.HIGHEST)
        sk_ref[r:r + 1, :] = lax.dot_general(
            k1, xw_r, dn, preferred_element_type=jnp.float32,
            precision=lax.Precision.HIGHEST)


def _dense0_body(x_ref, att_ref, basis_ref, q_ref, k_ref,
                 xw_ref, sq_ref, sk_ref):
    _dense_core(x_ref[...], att_ref, basis_ref, q_ref, k_ref,
                xw_ref, sq_ref, sk_ref)


def _dense1_body(p_ref, b_ref, att_ref, basis_ref, q_ref, k_ref,
                 xw_ref, sq_ref, sk_ref):
    h = jnp.maximum(p_ref[0] + p_ref[1] + b_ref[...], 0.0)
    _dense_core(h, att_ref, basis_ref, q_ref, k_ref, xw_ref, sq_ref, sk_ref)


_DENSE_OUT = [
    jax.ShapeDtypeStruct((_R, _N, _D), jnp.float32),
    jax.ShapeDtypeStruct((_R, _N), jnp.float32),
    jax.ShapeDtypeStruct((_R, _N), jnp.float32),
]
_DENSE_OUT_SPECS = [
    pl.BlockSpec((_R, _NBLK, _D), lambda i: (0, i, 0)),
    pl.BlockSpec((_R, _NBLK), lambda i: (0, i)),
    pl.BlockSpec((_R, _NBLK), lambda i: (0, i)),
]
_W_SPECS = [
    pl.BlockSpec((_R, _NBASES), lambda i: (0, 0), memory_space=pltpu.SMEM),
    pl.BlockSpec((_NBASES, _D, _D), lambda i: (0, 0, 0)),
    pl.BlockSpec((1, _D), lambda i: (0, 0)),
    pl.BlockSpec((1, _D), lambda i: (0, 0)),
]


def _dense0(x, att, basis, q1, k1):
    return pl.pallas_call(
        _dense0_body,
        grid=(_GRID,),
        in_specs=[pl.BlockSpec((_NBLK, _D), lambda i: (i, 0))] + _W_SPECS,
        out_specs=_DENSE_OUT_SPECS,
        out_shape=_DENSE_OUT,
    )(x, att, basis, q1, k1)


def _dense1(p, b1, att, basis, q1, k1):
    return pl.pallas_call(
        _dense1_body,
        grid=(_GRID,),
        in_specs=[pl.BlockSpec((_NC, _NBLK, _D), lambda i: (0, i, 0)),
                  pl.BlockSpec((1, _D), lambda i: (0, 0))] + _W_SPECS,
        out_specs=_DENSE_OUT_SPECS,
        out_shape=_DENSE_OUT,
    )(p, b1, att, basis, q1, k1)


def _combine_body(p_ref, b_ref, o_ref):
    o_ref[...] = p_ref[0] + p_ref[1] + b_ref[...]


def _combine(p, b1):
    return pl.pallas_call(
        _combine_body,
        grid=(_GRID,),
        in_specs=[pl.BlockSpec((_NC, _NBLK, _D), lambda i: (0, i, 0)),
                  pl.BlockSpec((1, _D), lambda i: (0, 0))],
        out_specs=pl.BlockSpec((_NBLK, _D), lambda i: (i, 0)),
        out_shape=jax.ShapeDtypeStruct((_N, _D), jnp.float32),
    )(p, b1)


_MESH = plsc.VectorSubcoreMesh(
    core_axis_name="c", subcore_axis_name="s",
    num_cores=_NC, num_subcores=_NS)


def _worker_base():
    cid = lax.axis_index("c")
    sid = lax.axis_index("s")
    wid = sid * _NC + cid
    base = pl.multiple_of(wid * _EPW, 8)
    return cid, sid, base


def _s1_body(sqf, skf, srcg, dstg, et, ex_out, den_out,
             table, eta, nbuf, acc, dst2d, den_s):
    cid, sid, base = _worker_base()
    zi = jnp.zeros((_LANE,), jnp.int32)
    zf = jnp.zeros((_LANE,), jnp.float32)

    pltpu.sync_copy(et.at[pl.ds(base, _EPW)], eta.at[pl.ds(0, _EPW)])
    pltpu.sync_copy(dstg.at[pl.ds(base, _EPW)], nbuf.at[pl.ds(0, _EPW)])
    for t in range(_EPW // _LANE, _EPAD // _LANE):
        eta[pl.ds(t * _LANE, _LANE)] = zi
        nbuf[pl.ds(t * _LANE, _LANE)] = zi

    pltpu.sync_copy(sqf, table)

    @pl.loop(0, _EPAD // _LANE)
    def _p1(i):
        o = pl.multiple_of(i * _LANE, _LANE)
        idx = eta[pl.ds(o, _LANE)] * _N + nbuf[pl.ds(o, _LANE)]
        acc[pl.ds(o, _LANE)] = plsc.load_gather(table, [idx])

    pltpu.sync_copy(srcg.at[pl.ds(base, _EPW)], nbuf.at[pl.ds(0, _EPW)])
    pltpu.sync_copy(skf, table)

    @pl.loop(0, _EPAD // _LANE)
    def _p2(i):
        o = pl.multiple_of(i * _LANE, _LANE)
        idx = eta[pl.ds(o, _LANE)] * _N + nbuf[pl.ds(o, _LANE)]
        kj = plsc.load_gather(table, [idx])
        z = acc[pl.ds(o, _LANE)] + kj
        al = jnp.maximum(z, 0.0) + _NEG * jnp.minimum(z, 0.0)
        acc[pl.ds(o, _LANE)] = jnp.exp(al)

    pltpu.sync_copy(acc.at[pl.ds(0, _EPW)], ex_out.at[pl.ds(base, _EPW)])
    for t in range(_EPW // _LANE, _EPAD // _LANE):
        acc[pl.ds(t * _LANE, _LANE)] = zf

    @pl.loop(0, _NCH - 1)
    def _ld(j):
        off = pl.multiple_of(base + j * _CH, 8)
        pltpu.sync_copy(dstg.at[pl.ds(off, _CH)], dst2d.at[j])
    lastoff = pl.multiple_of(base + (_NCH - 1) * _CH, 8)
    pltpu.sync_copy(dstg.at[pl.ds(lastoff, _TAIL)],
                    dst2d.at[_NCH - 1, pl.ds(0, _TAIL)])
    for t in range(_TAIL // _LANE, _CH // _LANE):
        dst2d[_NCH - 1, pl.ds(t * _LANE, _LANE)] = zi

    @pl.when(sid == 0)
    def _z():
        @pl.loop(0, _DNP // _LANE)
        def _zz(i):
            table[pl.ds(i * _LANE, _LANE)] = zf
        pltpu.sync_copy(table.at[pl.ds(0, _DNP)], den_s)

    plsc.subcore_barrier()

    @pl.loop(0, _NCH)
    def _sc(j):
        pltpu.sync_copy(acc.at[pl.ds(j * _CH, _CH)],
                        den_s.at[dst2d.at[j]], add=True)

    plsc.subcore_barrier()

    @pl.when(sid == 0)
    def _exp():
        pltpu.sync_copy(den_s, den_out.at[cid])


def _s1(sqf, skf, srcg, dstg, et):
    f = pl.kernel(
        _s1_body,
        out_type=[
            jax.ShapeDtypeStruct((_E,), jnp.float32),
            jax.ShapeDtypeStruct((_NC, _DNP), jnp.float32),
        ],
        mesh=_MESH,
        compiler_params=pltpu.CompilerParams(needs_layout_passes=False),
        scratch_types=[
            pltpu.VMEM((_R * _N,), jnp.float32),
            pltpu.VMEM((_EPAD,), jnp.int32),
            pltpu.VMEM((_EPAD,), jnp.int32),
            pltpu.VMEM((_EPAD,), jnp.float32),
            pltpu.VMEM((_NCH, _CH), jnp.int32),
            pltpu.VMEM_SHARED((_DNP,), jnp.float32),
        ],
    )
    return f(sqf, skf, srcg, dstg, et)


def _s2_body(xwf, ex, denp, srcg, dstg, et, out_p,
             den, dst2d, rb0, rb1, tbuf, sbuf, xbuf, ab0, ab1,
             gx0, gx1, outacc, gsem0, gsem1, ssem0, ssem1):
    cid, sid, base = _worker_base()
    zi = jnp.zeros((_LANE,), jnp.int32)
    zf = jnp.zeros((_LANE,), jnp.float32)

    pltpu.sync_copy(denp.at[0], den)

    @pl.loop(0, _DNP // _CH2)
    def _dsum(j):
        off = pl.multiple_of(j * _CH2, 8)
        pltpu.sync_copy(denp.at[1, pl.ds(off, _CH2)], xbuf)
        for s in range(_CH2 // _LANE):
            o = s * _LANE
            den[pl.ds(j * _CH2 + o, _LANE)] = (
                den[pl.ds(j * _CH2 + o, _LANE)] + xbuf[pl.ds(o, _LANE)])

    @pl.loop(0, _NCH2 - 1)
    def _ld(j):
        off = pl.multiple_of(base + j * _CH2, 8)
        pltpu.sync_copy(dstg.at[pl.ds(off, _CH2)], dst2d.at[j])
    lastoff = pl.multiple_of(base + (_NCH2 - 1) * _CH2, 8)
    pltpu.sync_copy(dstg.at[pl.ds(lastoff, _TAIL2)],
                    dst2d.at[_NCH2 - 1, pl.ds(0, _TAIL2)])
    for t in range(_TAIL2 // _LANE, _CH2 // _LANE):
        dst2d[_NCH2 - 1, pl.ds(t * _LANE, _LANE)] = zi

    @pl.loop(0, _CH2)
    def _zr(r):
        for s in range(_D // _LANE):
            rb0[r, pl.ds(s * _LANE, _LANE)] = zf

    stripe = pl.multiple_of(sid * _SW, 8)
    for jj in range(8):
        pltpu.sync_copy(rb0, outacc.at[pl.ds(stripe + jj * _CH2, _CH2)])

    @pl.when(sid < _NS - 1)
    def _zt0():
        pltpu.sync_copy(rb0, outacc.at[pl.ds(stripe + 8 * _CH2, _CH2)])
        pltpu.sync_copy(rb0.at[pl.ds(0, _SW - 9 * _CH2)],
                        outacc.at[pl.ds(stripe + 9 * _CH2, _SW - 9 * _CH2)])

    @pl.when(sid == _NS - 1)
    def _zt1():
        pltpu.sync_copy(rb0.at[pl.ds(0, _SWL - 8 * _CH2)],
                        outacc.at[pl.ds(stripe + 8 * _CH2, _SWL - 8 * _CH2)])

    plsc.subcore_barrier()

    def _bgidx(c, gx, ab):
        off = pl.multiple_of(base + c * _CH2, 8)
        pltpu.sync_copy(et.at[pl.ds(off, _CH2)], tbuf)
        pltpu.sync_copy(srcg.at[pl.ds(off, _CH2)], sbuf)
        pltpu.sync_copy(ex.at[pl.ds(off, _CH2)], xbuf)
        for s in range(_CH2 // _LANE):
            o = s * _LANE
            gx[pl.ds(o, _LANE)] = (tbuf[pl.ds(o, _LANE)] * _N
                                   + sbuf[pl.ds(o, _LANE)])
            d = dst2d[c, pl.ds(o, _LANE)]
            dn = plsc.load_gather(den, [d])
            ab[pl.ds(o, _LANE)] = xbuf[pl.ds(o, _LANE)] / (dn + 1e-16)

    def _scale(rb_s, ab):
        @pl.loop(0, _CH2)
        def _row(r):
            abc = plsc.load_gather(ab, [jnp.full((_LANE,), r, jnp.int32)])
            for s in range(_D // _LANE):
                rb_s[r, pl.ds(s * _LANE, _LANE)] = (
                    rb_s[r, pl.ds(s * _LANE, _LANE)] * abc)

    def _sdrain(c, rb_s, sem):
        pltpu.make_async_copy(rb_s, outacc.at[dst2d.at[c]], sem).wait()

    _bgidx(0, gx0, ab0)
    pltpu.async_copy(xwf.at[gx0], rb0, gsem0)
    _bgidx(1, gx1, ab1)
    pltpu.async_copy(xwf.at[gx1], rb1, gsem1)

    _NPAIR = (_NCH2 - 1) // 2

    @pl.loop(0, _NPAIR)
    def _pipe(p):
        c0 = p * 2
        c1 = c0 + 1
        pltpu.make_async_copy(xwf.at[gx0], rb0, gsem0).wait()
        _scale(rb0, ab0)
        pltpu.async_copy(rb0, outacc.at[dst2d.at[c0]], ssem0, add=True)
        pltpu.make_async_copy(xwf.at[gx1], rb1, gsem1).wait()
        _scale(rb1, ab1)
        pltpu.async_copy(rb1, outacc.at[dst2d.at[c1]], ssem1, add=True)

        @pl.when(p < _NPAIR - 1)
        def _pref():
            _sdrain(c0, rb0, ssem0)
            _bgidx(c0 + 2, gx0, ab0)
            pltpu.async_copy(xwf.at[gx0], rb0, gsem0)
            _sdrain(c1, rb1, ssem1)
            _bgidx(c1 + 2, gx1, ab1)
            pltpu.async_copy(xwf.at[gx1], rb1, gsem1)

    _sdrain(2 * _NPAIR - 2, rb0, ssem0)
    _sdrain(2 * _NPAIR - 1, rb1, ssem1)

    pltpu.sync_copy(et.at[pl.ds(lastoff, _TAIL2)], tbuf.at[pl.ds(0, _TAIL2)])
    pltpu.sync_copy(srcg.at[pl.ds(lastoff, _TAIL2)], sbuf.at[pl.ds(0, _TAIL2)])
    pltpu.sync_copy(ex.at[pl.ds(lastoff, _TAIL2)], xbuf.at[pl.ds(0, _TAIL2)])
    for s in range(_TAIL2 // _LANE):
        o = s * _LANE
        gx0[pl.ds(o, _LANE)] = (tbuf[pl.ds(o, _LANE)] * _N
                                + sbuf[pl.ds(o, _LANE)])
        d = dst2d[_NCH2 - 1, pl.ds(o, _LANE)]
        dn = plsc.load_gather(den, [d])
        ab0[pl.ds(o, _LANE)] = xbuf[pl.ds(o, _LANE)] / (dn + 1e-16)
    for t in range(_TAIL2 // _LANE, _CH2 // _LANE):
        gx0[pl.ds(t * _LANE, _LANE)] = zi
        ab0[pl.ds(t * _LANE, _LANE)] = zf
    pltpu.sync_copy(xwf.at[gx0], rb0)
    _scale(rb0, ab0)
    pltpu.sync_copy(rb0, outacc.at[dst2d.at[_NCH2 - 1]], add=True)

    plsc.subcore_barrier()

    for jj in range(4):
        off = pl.multiple_of(stripe + jj * _CH, 8)
        pltpu.sync_copy(outacc.at[pl.ds(off, _CH)],
                        out_p.at[cid, pl.ds(off, _CH)])
    toff = pl.multiple_of(stripe + 4 * _CH, 8)

    @pl.when(sid < _NS - 1)
    def _ex0():
        pltpu.sync_copy(outacc.at[pl.ds(toff, _SW - 4 * _CH)],
                        out_p.at[cid, pl.ds(toff, _SW - 4 * _CH)])

    @pl.when(sid == _NS - 1)
    def _ex1():
        pltpu.sync_copy(outacc.at[pl.ds(toff, _SWL - 4 * _CH)],
                        out_p.at[cid, pl.ds(toff, _SWL - 4 * _CH)])


def _s2(xwf, ex, denp, srcg, dstg, et):
    f = pl.kernel(
        _s2_body,
        out_type=jax.ShapeDtypeStruct((_NC, _N, _D), jnp.float32),
        mesh=_MESH,
        compiler_params=pltpu.CompilerParams(needs_layout_passes=False),
        scratch_types=[
            pltpu.VMEM((_DNP,), jnp.float32),
            pltpu.VMEM((_NCH2, _CH2), jnp.int32),
            pltpu.VMEM((_CH2, _D), jnp.float32),
            pltpu.VMEM((_CH2, _D), jnp.float32),
            pltpu.VMEM((_CH2,), jnp.int32),
            pltpu.VMEM((_CH2,), jnp.int32),
            pltpu.VMEM((_CH2,), jnp.float32),
            pltpu.VMEM((_CH2,), jnp.float32),
            pltpu.VMEM((_CH2,), jnp.float32),
            pltpu.VMEM((_CH2,), jnp.int32),
            pltpu.VMEM((_CH2,), jnp.int32),
            pltpu.VMEM_SHARED((_N, _D), jnp.float32),
            pltpu.SemaphoreType.DMA,
            pltpu.SemaphoreType.DMA,
            pltpu.SemaphoreType.DMA,
            pltpu.SemaphoreType.DMA,
        ],
    )
    return f(xwf, ex, denp, srcg, dstg, et)


def _layer0(x, srcg, dstg, et, att, basis, q, k):
    xw, sq, sk = _dense0(x, att, basis, q.reshape(1, _D), k.reshape(1, _D))
    ex, denp = _s1(sq.reshape(-1), sk.reshape(-1), srcg, dstg, et)
    return _s2(xw.reshape(_R * _N, _D), ex, denp, srcg, dstg, et)


def _layer(p, bprev, srcg, dstg, et, att, basis, q, k):
    xw, sq, sk = _dense1(p, bprev.reshape(1, _D), att, basis,
                         q.reshape(1, _D), k.reshape(1, _D))
    ex, denp = _s1(sq.reshape(-1), sk.reshape(-1), srcg, dstg, et)
    return _s2(xw.reshape(_R * _N, _D), ex, denp, srcg, dstg, et)


def kernel(x, edge_index, edge_type, att0, basis0, q0, k0, b0,
           att1, basis1, q1, k1, b1, att2, basis2, q2, k2, b2):
    srcg = edge_index[0]
    dstg = edge_index[1]
    et = edge_type
    p = _layer0(x, srcg, dstg, et, att0, basis0, q0, k0)
    p = _layer(p, b0, srcg, dstg, et, att1, basis1, q1, k1)
    p = _layer(p, b1, srcg, dstg, et, att2, basis2, q2, k2)
    return _combine(p, b2.reshape(1, _D))


# speedup vs baseline: 27.3026x; 1.3554x over previous
import jax
import jax.numpy as jnp
from jax import lax
from jax.experimental import pallas as pl
from jax.experimental.pallas import tpu as pltpu
from jax.experimental.pallas import tpu_sc as plsc

_N = 10000
_E = 320000
_R = 8
_NBASES = 4
_D = 128
_NEG = 0.2

_NC = 2
_NS = 16
_NW = _NC * _NS
_EPW = _E // _NW
_CH = 128
_NCH = (_EPW + _CH - 1) // _CH
_EPAD = _NCH * _CH
_TAIL = _EPW - (_NCH - 1) * _CH
_LANE = 16

_NBLK = 512
_GRID = (_N + _NBLK - 1) // _NBLK

_SW = 632
_SWL = _N - _SW * (_NS - 1)
_DNP = 10112

_CH2 = 64
_NCH2 = (_EPW + _CH2 - 1) // _CH2
_TAIL2 = _EPW - (_NCH2 - 1) * _CH2


def _dense_core(h, att_ref, basis_ref, q_ref, k_ref, xw_ref, sq_ref, sk_ref):
    q1 = q_ref[...]
    k1 = k_ref[...]
    dn = (((1,), (1,)), ((), ()))
    for r in range(_R):
        w_r = att_ref[r, 0] * basis_ref[0]
        for b in range(1, _NBASES):
            w_r = w_r + att_ref[r, b] * basis_ref[b]
        xw_r = jnp.dot(h, w_r, preferred_element_type=jnp.float32,
                       precision=lax.Precision.HIGHEST)
        xw_ref[r] = xw_r
        sq_ref[r:r + 1, :] = lax.dot_general(
            q1, xw_r, dn, preferred_element_type=jnp.float32,
            precision=lax.Precision.HIGHEST)
        sk_ref[r:r + 1, :] = lax.dot_general(
            k1, xw_r, dn, preferred_element_type=jnp.float32,
            precision=lax.Precision.HIGHEST)


def _dense0_body(x_ref, att_ref, basis_ref, q_ref, k_ref,
                 xw_ref, sq_ref, sk_ref):
    _dense_core(x_ref[...], att_ref, basis_ref, q_ref, k_ref,
                xw_ref, sq_ref, sk_ref)


def _dense1_body(p_ref, b_ref, att_ref, basis_ref, q_ref, k_ref,
                 xw_ref, sq_ref, sk_ref):
    h = jnp.maximum(p_ref[0] + p_ref[1] + b_ref[...], 0.0)
    _dense_core(h, att_ref, basis_ref, q_ref, k_ref, xw_ref, sq_ref, sk_ref)


_DENSE_OUT = [
    jax.ShapeDtypeStruct((_R, _N, _D), jnp.float32),
    jax.ShapeDtypeStruct((_R, _N), jnp.float32),
    jax.ShapeDtypeStruct((_R, _N), jnp.float32),
]
_DENSE_OUT_SPECS = [
    pl.BlockSpec((_R, _NBLK, _D), lambda i: (0, i, 0)),
    pl.BlockSpec((_R, _NBLK), lambda i: (0, i)),
    pl.BlockSpec((_R, _NBLK), lambda i: (0, i)),
]
_W_SPECS = [
    pl.BlockSpec((_R, _NBASES), lambda i: (0, 0), memory_space=pltpu.SMEM),
    pl.BlockSpec((_NBASES, _D, _D), lambda i: (0, 0, 0)),
    pl.BlockSpec((1, _D), lambda i: (0, 0)),
    pl.BlockSpec((1, _D), lambda i: (0, 0)),
]


def _dense0(x, att, basis, q1, k1):
    return pl.pallas_call(
        _dense0_body,
        grid=(_GRID,),
        in_specs=[pl.BlockSpec((_NBLK, _D), lambda i: (i, 0))] + _W_SPECS,
        out_specs=_DENSE_OUT_SPECS,
        out_shape=_DENSE_OUT,
    )(x, att, basis, q1, k1)


def _dense1(p, b1, att, basis, q1, k1):
    return pl.pallas_call(
        _dense1_body,
        grid=(_GRID,),
        in_specs=[pl.BlockSpec((_NC, _NBLK, _D), lambda i: (0, i, 0)),
                  pl.BlockSpec((1, _D), lambda i: (0, 0))] + _W_SPECS,
        out_specs=_DENSE_OUT_SPECS,
        out_shape=_DENSE_OUT,
    )(p, b1, att, basis, q1, k1)


def _combine_body(p_ref, b_ref, o_ref):
    o_ref[...] = p_ref[0] + p_ref[1] + b_ref[...]


def _combine(p, b1):
    return pl.pallas_call(
        _combine_body,
        grid=(_GRID,),
        in_specs=[pl.BlockSpec((_NC, _NBLK, _D), lambda i: (0, i, 0)),
                  pl.BlockSpec((1, _D), lambda i: (0, 0))],
        out_specs=pl.BlockSpec((_NBLK, _D), lambda i: (i, 0)),
        out_shape=jax.ShapeDtypeStruct((_N, _D), jnp.float32),
    )(p, b1)


def _densum_body(dp_ref, o_ref):
    o_ref[...] = dp_ref[0:1, :] + dp_ref[1:2, :]


def _densum(denp):
    out = pl.pallas_call(
        _densum_body,
        out_shape=jax.ShapeDtypeStruct((1, _DNP), jnp.float32),
    )(denp)
    return out.reshape(-1)


_MESH = plsc.VectorSubcoreMesh(
    core_axis_name="c", subcore_axis_name="s",
    num_cores=_NC, num_subcores=_NS)


def _worker_base():
    cid = lax.axis_index("c")
    sid = lax.axis_index("s")
    wid = sid * _NC + cid
    base = pl.multiple_of(wid * _EPW, 8)
    return cid, sid, base


def _s1_body(sqf, skf, srcg, dstg, et, ex_out, den_out, gidx_out,
             table, eta, nbuf, acc, dst2d, den_s):
    cid, sid, base = _worker_base()
    zi = jnp.zeros((_LANE,), jnp.int32)
    zf = jnp.zeros((_LANE,), jnp.float32)

    pltpu.sync_copy(et.at[pl.ds(base, _EPW)], eta.at[pl.ds(0, _EPW)])
    pltpu.sync_copy(dstg.at[pl.ds(base, _EPW)], nbuf.at[pl.ds(0, _EPW)])
    for t in range(_EPW // _LANE, _EPAD // _LANE):
        eta[pl.ds(t * _LANE, _LANE)] = zi
        nbuf[pl.ds(t * _LANE, _LANE)] = zi

    pltpu.sync_copy(sqf, table)

    @pl.loop(0, _EPAD // _LANE)
    def _p1(i):
        o = pl.multiple_of(i * _LANE, _LANE)
        idx = eta[pl.ds(o, _LANE)] * _N + nbuf[pl.ds(o, _LANE)]
        acc[pl.ds(o, _LANE)] = plsc.load_gather(table, [idx])

    pltpu.sync_copy(srcg.at[pl.ds(base, _EPW)], nbuf.at[pl.ds(0, _EPW)])
    pltpu.sync_copy(skf, table)

    @pl.loop(0, _EPAD // _LANE)
    def _p2(i):
        o = pl.multiple_of(i * _LANE, _LANE)
        idx = eta[pl.ds(o, _LANE)] * _N + nbuf[pl.ds(o, _LANE)]
        kj = plsc.load_gather(table, [idx])
        z = acc[pl.ds(o, _LANE)] + kj
        al = jnp.maximum(z, 0.0) + _NEG * jnp.minimum(z, 0.0)
        acc[pl.ds(o, _LANE)] = jnp.exp(al)
        eta[pl.ds(o, _LANE)] = idx

    pltpu.sync_copy(acc.at[pl.ds(0, _EPW)], ex_out.at[pl.ds(base, _EPW)])
    pltpu.sync_copy(eta.at[pl.ds(0, _EPW)], gidx_out.at[pl.ds(base, _EPW)])
    for t in range(_EPW // _LANE, _EPAD // _LANE):
        acc[pl.ds(t * _LANE, _LANE)] = zf

    @pl.loop(0, _NCH - 1)
    def _ld(j):
        off = pl.multiple_of(base + j * _CH, 8)
        pltpu.sync_copy(dstg.at[pl.ds(off, _CH)], dst2d.at[j])
    lastoff = pl.multiple_of(base + (_NCH - 1) * _CH, 8)
    pltpu.sync_copy(dstg.at[pl.ds(lastoff, _TAIL)],
                    dst2d.at[_NCH - 1, pl.ds(0, _TAIL)])
    for t in range(_TAIL // _LANE, _CH // _LANE):
        dst2d[_NCH - 1, pl.ds(t * _LANE, _LANE)] = zi

    @pl.when(sid == 0)
    def _z():
        @pl.loop(0, _DNP // _LANE)
        def _zz(i):
            table[pl.ds(i * _LANE, _LANE)] = zf
        pltpu.sync_copy(table.at[pl.ds(0, _DNP)], den_s)

    plsc.subcore_barrier()

    @pl.loop(0, _NCH)
    def _sc(j):
        pltpu.sync_copy(acc.at[pl.ds(j * _CH, _CH)],
                        den_s.at[dst2d.at[j]], add=True)

    plsc.subcore_barrier()

    @pl.when(sid == 0)
    def _exp():
        pltpu.sync_copy(den_s, den_out.at[cid])


def _s1(sqf, skf, srcg, dstg, et):
    f = pl.kernel(
        _s1_body,
        out_type=[
            jax.ShapeDtypeStruct((_E,), jnp.float32),
            jax.ShapeDtypeStruct((_NC, _DNP), jnp.float32),
            jax.ShapeDtypeStruct((_E,), jnp.int32),
        ],
        mesh=_MESH,
        compiler_params=pltpu.CompilerParams(needs_layout_passes=False),
        scratch_types=[
            pltpu.VMEM((_R * _N,), jnp.float32),
            pltpu.VMEM((_EPAD,), jnp.int32),
            pltpu.VMEM((_EPAD,), jnp.int32),
            pltpu.VMEM((_EPAD,), jnp.float32),
            pltpu.VMEM((_NCH, _CH), jnp.int32),
            pltpu.VMEM_SHARED((_DNP,), jnp.float32),
        ],
    )
    return f(sqf, skf, srcg, dstg, et)


def _s2_body(xwf, ex, gidx, dent, dstg, out_p,
             den, dst2d, rb0, rb1, xbuf, ab0, ab1,
             gx0, gx1, outacc, gsem0, gsem1, ssem0, ssem1):
    cid, sid, base = _worker_base()
    zi = jnp.zeros((_LANE,), jnp.int32)
    zf = jnp.zeros((_LANE,), jnp.float32)

    pltpu.sync_copy(dent, den)

    @pl.loop(0, _NCH2 - 1)
    def _ld(j):
        off = pl.multiple_of(base + j * _CH2, 8)
        pltpu.async_copy(dstg.at[pl.ds(off, _CH2)], dst2d.at[j], gsem0)

    @pl.loop(0, _NCH2 - 1)
    def _ldw(j):
        off = pl.multiple_of(base + j * _CH2, 8)
        pltpu.make_async_copy(dstg.at[pl.ds(off, _CH2)], dst2d.at[j],
                              gsem0).wait()
    lastoff = pl.multiple_of(base + (_NCH2 - 1) * _CH2, 8)
    pltpu.sync_copy(dstg.at[pl.ds(lastoff, _TAIL2)],
                    dst2d.at[_NCH2 - 1, pl.ds(0, _TAIL2)])
    for t in range(_TAIL2 // _LANE, _CH2 // _LANE):
        dst2d[_NCH2 - 1, pl.ds(t * _LANE, _LANE)] = zi

    @pl.loop(0, _CH2)
    def _zr(r):
        for s in range(_D // _LANE):
            rb0[r, pl.ds(s * _LANE, _LANE)] = zf

    stripe = pl.multiple_of(sid * _SW, 8)
    for jj in range(8):
        pltpu.sync_copy(rb0, outacc.at[pl.ds(stripe + jj * _CH2, _CH2)])

    @pl.when(sid < _NS - 1)
    def _zt0():
        pltpu.sync_copy(rb0, outacc.at[pl.ds(stripe + 8 * _CH2, _CH2)])
        pltpu.sync_copy(rb0.at[pl.ds(0, _SW - 9 * _CH2)],
                        outacc.at[pl.ds(stripe + 9 * _CH2, _SW - 9 * _CH2)])

    @pl.when(sid == _NS - 1)
    def _zt1():
        pltpu.sync_copy(rb0.at[pl.ds(0, _SWL - 8 * _CH2)],
                        outacc.at[pl.ds(stripe + 8 * _CH2, _SWL - 8 * _CH2)])

    plsc.subcore_barrier()

    def _bgidx(c, gx, ab):
        off = pl.multiple_of(base + c * _CH2, 8)
        pltpu.sync_copy(gidx.at[pl.ds(off, _CH2)], gx)
        pltpu.sync_copy(ex.at[pl.ds(off, _CH2)], xbuf)
        for s in range(_CH2 // _LANE):
            o = s * _LANE
            d = dst2d[c, pl.ds(o, _LANE)]
            dn = plsc.load_gather(den, [d])
            ab[pl.ds(o, _LANE)] = xbuf[pl.ds(o, _LANE)] / (dn + 1e-16)

    def _scale(rb_s, ab):
        @pl.loop(0, _CH2, unroll=8)
        def _row(r):
            abc = plsc.load_gather(ab, [jnp.full((_LANE,), r, jnp.int32)])
            for s in range(_D // _LANE):
                rb_s[r, pl.ds(s * _LANE, _LANE)] = (
                    rb_s[r, pl.ds(s * _LANE, _LANE)] * abc)

    def _sdrain(c, rb_s, sem):
        pltpu.make_async_copy(rb_s, outacc.at[dst2d.at[c]], sem).wait()

    _bgidx(0, gx0, ab0)
    pltpu.async_copy(xwf.at[gx0], rb0, gsem0)
    _bgidx(1, gx1, ab1)
    pltpu.async_copy(xwf.at[gx1], rb1, gsem1)

    _NPAIR = (_NCH2 - 1) // 2

    @pl.loop(0, _NPAIR)
    def _pipe(p):
        c0 = p * 2
        c1 = c0 + 1
        pltpu.make_async_copy(xwf.at[gx0], rb0, gsem0).wait()
        _scale(rb0, ab0)
        pltpu.async_copy(rb0, outacc.at[dst2d.at[c0]], ssem0, add=True)
        pltpu.make_async_copy(xwf.at[gx1], rb1, gsem1).wait()
        _scale(rb1, ab1)
        pltpu.async_copy(rb1, outacc.at[dst2d.at[c1]], ssem1, add=True)

        @pl.when(p < _NPAIR - 1)
        def _pref():
            _sdrain(c0, rb0, ssem0)
            _bgidx(c0 + 2, gx0, ab0)
            pltpu.async_copy(xwf.at[gx0], rb0, gsem0)
            _sdrain(c1, rb1, ssem1)
            _bgidx(c1 + 2, gx1, ab1)
            pltpu.async_copy(xwf.at[gx1], rb1, gsem1)

    _sdrain(2 * _NPAIR - 2, rb0, ssem0)
    _sdrain(2 * _NPAIR - 1, rb1, ssem1)

    pltpu.sync_copy(gidx.at[pl.ds(lastoff, _TAIL2)],
                    gx0.at[pl.ds(0, _TAIL2)])
    pltpu.sync_copy(ex.at[pl.ds(lastoff, _TAIL2)], xbuf.at[pl.ds(0, _TAIL2)])
    for s in range(_TAIL2 // _LANE):
        o = s * _LANE
        d = dst2d[_NCH2 - 1, pl.ds(o, _LANE)]
        dn = plsc.load_gather(den, [d])
        ab0[pl.ds(o, _LANE)] = xbuf[pl.ds(o, _LANE)] / (dn + 1e-16)
    for t in range(_TAIL2 // _LANE, _CH2 // _LANE):
        gx0[pl.ds(t * _LANE, _LANE)] = zi
        ab0[pl.ds(t * _LANE, _LANE)] = zf
    pltpu.sync_copy(xwf.at[gx0], rb0)
    _scale(rb0, ab0)
    pltpu.sync_copy(rb0, outacc.at[dst2d.at[_NCH2 - 1]], add=True)

    plsc.subcore_barrier()

    for jj in range(4):
        off = pl.multiple_of(stripe + jj * _CH, 8)
        pltpu.sync_copy(outacc.at[pl.ds(off, _CH)],
                        out_p.at[cid, pl.ds(off, _CH)])
    toff = pl.multiple_of(stripe + 4 * _CH, 8)

    @pl.when(sid < _NS - 1)
    def _ex0():
        pltpu.sync_copy(outacc.at[pl.ds(toff, _SW - 4 * _CH)],
                        out_p.at[cid, pl.ds(toff, _SW - 4 * _CH)])

    @pl.when(sid == _NS - 1)
    def _ex1():
        pltpu.sync_copy(outacc.at[pl.ds(toff, _SWL - 4 * _CH)],
                        out_p.at[cid, pl.ds(toff, _SWL - 4 * _CH)])


def _s2(xwf, ex, gidx, dent, dstg):
    f = pl.kernel(
        _s2_body,
        out_type=jax.ShapeDtypeStruct((_NC, _N, _D), jnp.float32),
        mesh=_MESH,
        compiler_params=pltpu.CompilerParams(needs_layout_passes=False),
        scratch_types=[
            pltpu.VMEM((_DNP,), jnp.float32),
            pltpu.VMEM((_NCH2, _CH2), jnp.int32),
            pltpu.VMEM((_CH2, _D), jnp.float32),
            pltpu.VMEM((_CH2, _D), jnp.float32),
            pltpu.VMEM((_CH2,), jnp.float32),
            pltpu.VMEM((_CH2,), jnp.float32),
            pltpu.VMEM((_CH2,), jnp.float32),
            pltpu.VMEM((_CH2,), jnp.int32),
            pltpu.VMEM((_CH2,), jnp.int32),
            pltpu.VMEM_SHARED((_N, _D), jnp.float32),
            pltpu.SemaphoreType.DMA,
            pltpu.SemaphoreType.DMA,
            pltpu.SemaphoreType.DMA,
            pltpu.SemaphoreType.DMA,
        ],
    )
    return f(xwf, ex, gidx, dent, dstg)


def _layer0(x, srcg, dstg, et, att, basis, q, k):
    xw, sq, sk = _dense0(x, att, basis, q.reshape(1, _D), k.reshape(1, _D))
    ex, denp, gidx = _s1(sq.reshape(-1), sk.reshape(-1), srcg, dstg, et)
    return _s2(xw.reshape(_R * _N, _D), ex, gidx, _densum(denp), dstg)


def _layer(p, bprev, srcg, dstg, et, att, basis, q, k):
    xw, sq, sk = _dense1(p, bprev.reshape(1, _D), att, basis,
                         q.reshape(1, _D), k.reshape(1, _D))
    ex, denp, gidx = _s1(sq.reshape(-1), sk.reshape(-1), srcg, dstg, et)
    return _s2(xw.reshape(_R * _N, _D), ex, gidx, _densum(denp), dstg)


def kernel(x, edge_index, edge_type, att0, basis0, q0, k0, b0,
           att1, basis1, q1, k1, b1, att2, basis2, q2, k2, b2):
    srcg = edge_index[0]
    dstg = edge_index[1]
    et = edge_type
    p = _layer0(x, srcg, dstg, et, att0, basis0, q0, k0)
    p = _layer(p, b0, srcg, dstg, et, att1, basis1, q1, k1)
    p = _layer(p, b1, srcg, dstg, et, att2, basis2, q2, k2)
    return _combine(p, b2.reshape(1, _D))


# speedup vs baseline: 28.0393x; 1.0270x over previous
import jax
import jax.numpy as jnp
from jax import lax
from jax.experimental import pallas as pl
from jax.experimental.pallas import tpu as pltpu
from jax.experimental.pallas import tpu_sc as plsc

_N = 10000
_E = 320000
_R = 8
_NBASES = 4
_D = 128
_NEG = 0.2

_NC = 2
_NS = 16
_NW = _NC * _NS
_EPW = _E // _NW
_CH = 128
_NCH = (_EPW + _CH - 1) // _CH
_EPAD = _NCH * _CH
_TAIL = _EPW - (_NCH - 1) * _CH
_LANE = 16

_NBLK = 1024
_GRID = (_N + _NBLK - 1) // _NBLK

_SW = 632
_SWL = _N - _SW * (_NS - 1)
_DNP = 10112

_CH2 = 64
_NCH2 = (_EPW + _CH2 - 1) // _CH2
_TAIL2 = _EPW - (_NCH2 - 1) * _CH2


def _dense_core(h, att_ref, basis_ref, q_ref, k_ref, xw_ref, sq_ref, sk_ref):
    q1 = q_ref[...]
    k1 = k_ref[...]
    dn = (((1,), (1,)), ((), ()))
    for r in range(_R):
        w_r = att_ref[r, 0] * basis_ref[0]
        for b in range(1, _NBASES):
            w_r = w_r + att_ref[r, b] * basis_ref[b]
        xw_r = jnp.dot(h, w_r, preferred_element_type=jnp.float32,
                       precision=lax.Precision.HIGHEST)
        xw_ref[r] = xw_r
        sq_ref[r:r + 1, :] = lax.dot_general(
            q1, xw_r, dn, preferred_element_type=jnp.float32,
            precision=lax.Precision.HIGHEST)
        sk_ref[r:r + 1, :] = lax.dot_general(
            k1, xw_r, dn, preferred_element_type=jnp.float32,
            precision=lax.Precision.HIGHEST)


def _dense0_body(x_ref, att_ref, basis_ref, q_ref, k_ref,
                 xw_ref, sq_ref, sk_ref):
    _dense_core(x_ref[...], att_ref, basis_ref, q_ref, k_ref,
                xw_ref, sq_ref, sk_ref)


def _dense1_body(p_ref, b_ref, att_ref, basis_ref, q_ref, k_ref,
                 xw_ref, sq_ref, sk_ref):
    h = jnp.maximum(p_ref[0] + p_ref[1] + b_ref[...], 0.0)
    _dense_core(h, att_ref, basis_ref, q_ref, k_ref, xw_ref, sq_ref, sk_ref)


_DENSE_OUT = [
    jax.ShapeDtypeStruct((_R, _N, _D), jnp.float32),
    jax.ShapeDtypeStruct((_R, _N), jnp.float32),
    jax.ShapeDtypeStruct((_R, _N), jnp.float32),
]
_DENSE_OUT_SPECS = [
    pl.BlockSpec((_R, _NBLK, _D), lambda i: (0, i, 0)),
    pl.BlockSpec((_R, _NBLK), lambda i: (0, i)),
    pl.BlockSpec((_R, _NBLK), lambda i: (0, i)),
]
_W_SPECS = [
    pl.BlockSpec((_R, _NBASES), lambda i: (0, 0), memory_space=pltpu.SMEM),
    pl.BlockSpec((_NBASES, _D, _D), lambda i: (0, 0, 0)),
    pl.BlockSpec((1, _D), lambda i: (0, 0)),
    pl.BlockSpec((1, _D), lambda i: (0, 0)),
]


def _dense0(x, att, basis, q1, k1):
    return pl.pallas_call(
        _dense0_body,
        grid=(_GRID,),
        in_specs=[pl.BlockSpec((_NBLK, _D), lambda i: (i, 0))] + _W_SPECS,
        out_specs=_DENSE_OUT_SPECS,
        out_shape=_DENSE_OUT,
    )(x, att, basis, q1, k1)


def _dense1(p, b1, att, basis, q1, k1):
    return pl.pallas_call(
        _dense1_body,
        grid=(_GRID,),
        in_specs=[pl.BlockSpec((_NC, _NBLK, _D), lambda i: (0, i, 0)),
                  pl.BlockSpec((1, _D), lambda i: (0, 0))] + _W_SPECS,
        out_specs=_DENSE_OUT_SPECS,
        out_shape=_DENSE_OUT,
    )(p, b1, att, basis, q1, k1)


def _combine_body(p_ref, b_ref, o_ref):
    o_ref[...] = p_ref[0] + p_ref[1] + b_ref[...]


def _combine(p, b1):
    return pl.pallas_call(
        _combine_body,
        grid=(_GRID,),
        in_specs=[pl.BlockSpec((_NC, _NBLK, _D), lambda i: (0, i, 0)),
                  pl.BlockSpec((1, _D), lambda i: (0, 0))],
        out_specs=pl.BlockSpec((_NBLK, _D), lambda i: (i, 0)),
        out_shape=jax.ShapeDtypeStruct((_N, _D), jnp.float32),
    )(p, b1)


def _densum_body(dp_ref, o_ref):
    o_ref[...] = dp_ref[0:1, :] + dp_ref[1:2, :]


def _densum(denp):
    out = pl.pallas_call(
        _densum_body,
        out_shape=jax.ShapeDtypeStruct((1, _DNP), jnp.float32),
    )(denp)
    return out.reshape(-1)


_MESH = plsc.VectorSubcoreMesh(
    core_axis_name="c", subcore_axis_name="s",
    num_cores=_NC, num_subcores=_NS)


def _worker_base():
    cid = lax.axis_index("c")
    sid = lax.axis_index("s")
    wid = sid * _NC + cid
    base = pl.multiple_of(wid * _EPW, 8)
    return cid, sid, base


def _s1_body(sqf, skf, srcg, dstg, et, ex_out, den_out, gidx_out,
             table, eta, nbuf, acc, dst2d, den_s):
    cid, sid, base = _worker_base()
    zi = jnp.zeros((_LANE,), jnp.int32)
    zf = jnp.zeros((_LANE,), jnp.float32)

    pltpu.sync_copy(et.at[pl.ds(base, _EPW)], eta.at[pl.ds(0, _EPW)])
    pltpu.sync_copy(dstg.at[pl.ds(base, _EPW)], nbuf.at[pl.ds(0, _EPW)])
    for t in range(_EPW // _LANE, _EPAD // _LANE):
        eta[pl.ds(t * _LANE, _LANE)] = zi
        nbuf[pl.ds(t * _LANE, _LANE)] = zi

    pltpu.sync_copy(sqf, table)

    @pl.loop(0, _EPAD // _LANE)
    def _p1(i):
        o = pl.multiple_of(i * _LANE, _LANE)
        idx = eta[pl.ds(o, _LANE)] * _N + nbuf[pl.ds(o, _LANE)]
        acc[pl.ds(o, _LANE)] = plsc.load_gather(table, [idx])

    pltpu.sync_copy(srcg.at[pl.ds(base, _EPW)], nbuf.at[pl.ds(0, _EPW)])
    pltpu.sync_copy(skf, table)

    @pl.loop(0, _EPAD // _LANE)
    def _p2(i):
        o = pl.multiple_of(i * _LANE, _LANE)
        idx = eta[pl.ds(o, _LANE)] * _N + nbuf[pl.ds(o, _LANE)]
        kj = plsc.load_gather(table, [idx])
        z = acc[pl.ds(o, _LANE)] + kj
        al = jnp.maximum(z, 0.0) + _NEG * jnp.minimum(z, 0.0)
        acc[pl.ds(o, _LANE)] = jnp.exp(al)
        eta[pl.ds(o, _LANE)] = idx

    pltpu.sync_copy(acc.at[pl.ds(0, _EPW)], ex_out.at[pl.ds(base, _EPW)])
    pltpu.sync_copy(eta.at[pl.ds(0, _EPW)], gidx_out.at[pl.ds(base, _EPW)])
    for t in range(_EPW // _LANE, _EPAD // _LANE):
        acc[pl.ds(t * _LANE, _LANE)] = zf

    @pl.loop(0, _NCH - 1)
    def _ld(j):
        off = pl.multiple_of(base + j * _CH, 8)
        pltpu.sync_copy(dstg.at[pl.ds(off, _CH)], dst2d.at[j])
    lastoff = pl.multiple_of(base + (_NCH - 1) * _CH, 8)
    pltpu.sync_copy(dstg.at[pl.ds(lastoff, _TAIL)],
                    dst2d.at[_NCH - 1, pl.ds(0, _TAIL)])
    for t in range(_TAIL // _LANE, _CH // _LANE):
        dst2d[_NCH - 1, pl.ds(t * _LANE, _LANE)] = zi

    @pl.when(sid == 0)
    def _z():
        @pl.loop(0, _DNP // _LANE)
        def _zz(i):
            table[pl.ds(i * _LANE, _LANE)] = zf
        pltpu.sync_copy(table.at[pl.ds(0, _DNP)], den_s)

    plsc.subcore_barrier()

    @pl.loop(0, _NCH)
    def _sc(j):
        pltpu.sync_copy(acc.at[pl.ds(j * _CH, _CH)],
                        den_s.at[dst2d.at[j]], add=True)

    plsc.subcore_barrier()

    @pl.when(sid == 0)
    def _exp():
        pltpu.sync_copy(den_s, den_out.at[cid])


def _s1(sqf, skf, srcg, dstg, et):
    f = pl.kernel(
        _s1_body,
        out_type=[
            jax.ShapeDtypeStruct((_E,), jnp.float32),
            jax.ShapeDtypeStruct((_NC, _DNP), jnp.float32),
            jax.ShapeDtypeStruct((_E,), jnp.int32),
        ],
        mesh=_MESH,
        compiler_params=pltpu.CompilerParams(needs_layout_passes=False),
        scratch_types=[
            pltpu.VMEM((_R * _N,), jnp.float32),
            pltpu.VMEM((_EPAD,), jnp.int32),
            pltpu.VMEM((_EPAD,), jnp.int32),
            pltpu.VMEM((_EPAD,), jnp.float32),
            pltpu.VMEM((_NCH, _CH), jnp.int32),
            pltpu.VMEM_SHARED((_DNP,), jnp.float32),
        ],
    )
    return f(sqf, skf, srcg, dstg, et)


def _s2_body(xwf, ex, gidx, dent, dstg, out_p,
             den, dst2d, rb0, rb1, xbuf, ab0, ab1,
             gx0, gx1, outacc, gsem0, gsem1, ssem0, ssem1):
    cid, sid, base = _worker_base()
    zi = jnp.zeros((_LANE,), jnp.int32)
    zf = jnp.zeros((_LANE,), jnp.float32)

    pltpu.sync_copy(dent, den)

    @pl.loop(0, _NCH2 - 1)
    def _ld(j):
        off = pl.multiple_of(base + j * _CH2, 8)
        pltpu.async_copy(dstg.at[pl.ds(off, _CH2)], dst2d.at[j], gsem0)

    @pl.loop(0, _NCH2 - 1)
    def _ldw(j):
        off = pl.multiple_of(base + j * _CH2, 8)
        pltpu.make_async_copy(dstg.at[pl.ds(off, _CH2)], dst2d.at[j],
                              gsem0).wait()
    lastoff = pl.multiple_of(base + (_NCH2 - 1) * _CH2, 8)
    pltpu.sync_copy(dstg.at[pl.ds(lastoff, _TAIL2)],
                    dst2d.at[_NCH2 - 1, pl.ds(0, _TAIL2)])
    for t in range(_TAIL2 // _LANE, _CH2 // _LANE):
        dst2d[_NCH2 - 1, pl.ds(t * _LANE, _LANE)] = zi

    @pl.loop(0, _CH2)
    def _zr(r):
        for s in range(_D // _LANE):
            rb0[r, pl.ds(s * _LANE, _LANE)] = zf

    stripe = pl.multiple_of(sid * _SW, 8)
    for jj in range(8):
        pltpu.sync_copy(rb0, outacc.at[pl.ds(stripe + jj * _CH2, _CH2)])

    @pl.when(sid < _NS - 1)
    def _zt0():
        pltpu.sync_copy(rb0, outacc.at[pl.ds(stripe + 8 * _CH2, _CH2)])
        pltpu.sync_copy(rb0.at[pl.ds(0, _SW - 9 * _CH2)],
                        outacc.at[pl.ds(stripe + 9 * _CH2, _SW - 9 * _CH2)])

    @pl.when(sid == _NS - 1)
    def _zt1():
        pltpu.sync_copy(rb0.at[pl.ds(0, _SWL - 8 * _CH2)],
                        outacc.at[pl.ds(stripe + 8 * _CH2, _SWL - 8 * _CH2)])

    plsc.subcore_barrier()

    def _bgidx(c, gx, ab):
        off = pl.multiple_of(base + c * _CH2, 8)
        pltpu.sync_copy(gidx.at[pl.ds(off, _CH2)], gx)
        pltpu.sync_copy(ex.at[pl.ds(off, _CH2)], xbuf)
        for s in range(_CH2 // _LANE):
            o = s * _LANE
            d = dst2d[c, pl.ds(o, _LANE)]
            dn = plsc.load_gather(den, [d])
            ab[pl.ds(o, _LANE)] = xbuf[pl.ds(o, _LANE)] / (dn + 1e-16)

    def _scale(rb_s, ab):
        @pl.loop(0, _CH2, unroll=8)
        def _row(r):
            abc = plsc.load_gather(ab, [jnp.full((_LANE,), r, jnp.int32)])
            for s in range(_D // _LANE):
                rb_s[r, pl.ds(s * _LANE, _LANE)] = (
                    rb_s[r, pl.ds(s * _LANE, _LANE)] * abc)

    def _sdrain(c, rb_s, sem):
        pltpu.make_async_copy(rb_s, outacc.at[dst2d.at[c]], sem).wait()

    _bgidx(0, gx0, ab0)
    pltpu.async_copy(xwf.at[gx0], rb0, gsem0)
    _bgidx(1, gx1, ab1)
    pltpu.async_copy(xwf.at[gx1], rb1, gsem1)

    _NPAIR = (_NCH2 - 1) // 2

    @pl.loop(0, _NPAIR)
    def _pipe(p):
        c0 = p * 2
        c1 = c0 + 1
        pltpu.make_async_copy(xwf.at[gx0], rb0, gsem0).wait()
        _scale(rb0, ab0)
        pltpu.async_copy(rb0, outacc.at[dst2d.at[c0]], ssem0, add=True)
        pltpu.make_async_copy(xwf.at[gx1], rb1, gsem1).wait()
        _scale(rb1, ab1)
        pltpu.async_copy(rb1, outacc.at[dst2d.at[c1]], ssem1, add=True)

        @pl.when(p < _NPAIR - 1)
        def _pref():
            _sdrain(c0, rb0, ssem0)
            _bgidx(c0 + 2, gx0, ab0)
            pltpu.async_copy(xwf.at[gx0], rb0, gsem0)
            _sdrain(c1, rb1, ssem1)
            _bgidx(c1 + 2, gx1, ab1)
            pltpu.async_copy(xwf.at[gx1], rb1, gsem1)

    _sdrain(2 * _NPAIR - 2, rb0, ssem0)
    _sdrain(2 * _NPAIR - 1, rb1, ssem1)

    pltpu.sync_copy(gidx.at[pl.ds(lastoff, _TAIL2)],
                    gx0.at[pl.ds(0, _TAIL2)])
    pltpu.sync_copy(ex.at[pl.ds(lastoff, _TAIL2)], xbuf.at[pl.ds(0, _TAIL2)])
    for s in range(_TAIL2 // _LANE):
        o = s * _LANE
        d = dst2d[_NCH2 - 1, pl.ds(o, _LANE)]
        dn = plsc.load_gather(den, [d])
        ab0[pl.ds(o, _LANE)] = xbuf[pl.ds(o, _LANE)] / (dn + 1e-16)
    for t in range(_TAIL2 // _LANE, _CH2 // _LANE):
        gx0[pl.ds(t * _LANE, _LANE)] = zi
        ab0[pl.ds(t * _LANE, _LANE)] = zf
    pltpu.sync_copy(xwf.at[gx0], rb0)
    _scale(rb0, ab0)
    pltpu.sync_copy(rb0, outacc.at[dst2d.at[_NCH2 - 1]], add=True)

    plsc.subcore_barrier()

    for jj in range(4):
        off = pl.multiple_of(stripe + jj * _CH, 8)
        pltpu.sync_copy(outacc.at[pl.ds(off, _CH)],
                        out_p.at[cid, pl.ds(off, _CH)])
    toff = pl.multiple_of(stripe + 4 * _CH, 8)

    @pl.when(sid < _NS - 1)
    def _ex0():
        pltpu.sync_copy(outacc.at[pl.ds(toff, _SW - 4 * _CH)],
                        out_p.at[cid, pl.ds(toff, _SW - 4 * _CH)])

    @pl.when(sid == _NS - 1)
    def _ex1():
        pltpu.sync_copy(outacc.at[pl.ds(toff, _SWL - 4 * _CH)],
                        out_p.at[cid, pl.ds(toff, _SWL - 4 * _CH)])


def _s2(xwf, ex, gidx, dent, dstg):
    f = pl.kernel(
        _s2_body,
        out_type=jax.ShapeDtypeStruct((_NC, _N, _D), jnp.float32),
        mesh=_MESH,
        compiler_params=pltpu.CompilerParams(needs_layout_passes=False),
        scratch_types=[
            pltpu.VMEM((_DNP,), jnp.float32),
            pltpu.VMEM((_NCH2, _CH2), jnp.int32),
            pltpu.VMEM((_CH2, _D), jnp.float32),
            pltpu.VMEM((_CH2, _D), jnp.float32),
            pltpu.VMEM((_CH2,), jnp.float32),
            pltpu.VMEM((_CH2,), jnp.float32),
            pltpu.VMEM((_CH2,), jnp.float32),
            pltpu.VMEM((_CH2,), jnp.int32),
            pltpu.VMEM((_CH2,), jnp.int32),
            pltpu.VMEM_SHARED((_N, _D), jnp.float32),
            pltpu.SemaphoreType.DMA,
            pltpu.SemaphoreType.DMA,
            pltpu.SemaphoreType.DMA,
            pltpu.SemaphoreType.DMA,
        ],
    )
    return f(xwf, ex, gidx, dent, dstg)


def _layer0(x, srcg, dstg, et, att, basis, q, k):
    xw, sq, sk = _dense0(x, att, basis, q.reshape(1, _D), k.reshape(1, _D))
    ex, denp, gidx = _s1(sq.reshape(-1), sk.reshape(-1), srcg, dstg, et)
    return _s2(xw.reshape(_R * _N, _D), ex, gidx, _densum(denp), dstg)


def _layer(p, bprev, srcg, dstg, et, att, basis, q, k):
    xw, sq, sk = _dense1(p, bprev.reshape(1, _D), att, basis,
                         q.reshape(1, _D), k.reshape(1, _D))
    ex, denp, gidx = _s1(sq.reshape(-1), sk.reshape(-1), srcg, dstg, et)
    return _s2(xw.reshape(_R * _N, _D), ex, gidx, _densum(denp), dstg)


def kernel(x, edge_index, edge_type, att0, basis0, q0, k0, b0,
           att1, basis1, q1, k1, b1, att2, basis2, q2, k2, b2):
    srcg = edge_index[0]
    dstg = edge_index[1]
    et = edge_type
    p = _layer0(x, srcg, dstg, et, att0, basis0, q0, k0)
    p = _layer(p, b0, srcg, dstg, et, att1, basis1, q1, k1)
    p = _layer(p, b1, srcg, dstg, et, att2, basis2, q2, k2)
    return _combine(p, b2.reshape(1, _D))
